# Initial kernel scaffold; baseline (speedup 1.0000x reference)
import jax
import jax.numpy as jnp
from jax.experimental import pallas as pl


def kernel(x, pos_edge_index, neg_edge_index, color_w0, color_b0, color_w1, color_b1, Cx, g_lin1_w, g_lin1_b, g_lin2_w, g_lin2_b, l0_w, l0_b, l1_w, l1_b):
    raise NotImplementedError("write your pallas kernel here")


# speedup vs baseline: 1.0060x; 1.0060x over previous
import functools

import jax
import jax.numpy as jnp
from jax.experimental import pallas as pl
from jax.experimental.pallas import tpu as pltpu

N = 10000
E = 320000
ROW_BLK = 1000


def _mp(x, ei):
    return jnp.zeros(x.shape, x.dtype).at[ei[1]].add(x[ei[0]])


def _local_body(a_ref, b_ref, c_ref, wa_ref, wb_ref, wc_ref, bias_ref, o_ref,
                *, activate):
    acc = jnp.dot(a_ref[...], wa_ref[...].T, preferred_element_type=jnp.float32)
    acc += jnp.dot(b_ref[...], wb_ref[...].T, preferred_element_type=jnp.float32)
    acc += jnp.dot(c_ref[...], wc_ref[...].T, preferred_element_type=jnp.float32)
    acc += bias_ref[...][None, :]
    if activate:
        acc = jnp.tanh(acc)
    o_ref[...] = acc


def _local_layer(a, b, c, w, bias, activate):
    d = a.shape[1]
    h = w.shape[0]
    wa, wb, wc = w[:, :d], w[:, d:2 * d], w[:, 2 * d:]
    grid = (N // ROW_BLK,)
    in_spec = pl.BlockSpec((ROW_BLK, d), lambda i: (i, 0))
    w_spec = pl.BlockSpec((h, d), lambda i: (0, 0))
    return pl.pallas_call(
        functools.partial(_local_body, activate=activate),
        grid=grid,
        in_specs=[in_spec, in_spec, in_spec, w_spec, w_spec, w_spec,
                  pl.BlockSpec((h,), lambda i: (0,))],
        out_specs=pl.BlockSpec((ROW_BLK, h), lambda i: (i, 0)),
        out_shape=jax.ShapeDtypeStruct((N, h), jnp.float32),
    )(a, b, c, wa, wb, wc, bias)


def _color_body(x_ref, w0_ref, b0_ref, w1_ref, b1_ref, cx_ref, o_ref):
    h = jnp.maximum(
        jnp.dot(x_ref[...], w0_ref[...].T, preferred_element_type=jnp.float32)
        + b0_ref[...][None, :], 0.0)
    cc = jnp.dot(h, w1_ref[...].T, preferred_element_type=jnp.float32) \
        + b1_ref[...][None, :]
    cx = jnp.dot(cc, cx_ref[...].T, preferred_element_type=jnp.float32)
    o_ref[...] = jax.nn.softmax(cx, axis=1)


def _color(x, w0, b0, w1, b1, cx):
    grid = (N // ROW_BLK,)
    return pl.pallas_call(
        _color_body,
        grid=grid,
        in_specs=[
            pl.BlockSpec((ROW_BLK, 128), lambda i: (i, 0)),
            pl.BlockSpec((64, 128), lambda i: (0, 0)),
            pl.BlockSpec((64,), lambda i: (0,)),
            pl.BlockSpec((16, 64), lambda i: (0, 0)),
            pl.BlockSpec((16,), lambda i: (0,)),
            pl.BlockSpec((3, 16), lambda i: (0, 0)),
        ],
        out_specs=pl.BlockSpec((ROW_BLK, 3), lambda i: (i, 0)),
        out_shape=jax.ShapeDtypeStruct((N, 3), jnp.float32),
    )(x, w0, b0, w1, b1, cx)


def kernel(x, pos_edge_index, neg_edge_index, color_w0, color_b0, color_w1,
           color_b1, Cx, g_lin1_w, g_lin1_b, g_lin2_w, g_lin2_b, l0_w, l0_b,
           l1_w, l1_b):
    G = _color(x, color_w0, color_b0, color_w1, color_b1, Cx)
    for i in range(5):
        g_pos = _mp(G, pos_edge_index)
        g_neg = _mp(G, neg_edge_index)
        cat = jnp.concatenate((G, g_pos, g_neg), axis=1)
        h = jnp.tanh(cat @ g_lin1_w[i].T + g_lin1_b[i])
        C = h @ g_lin2_w[i].T + g_lin2_b[i]
        G = jax.nn.softmax(C, axis=1)
    G = G @ Cx

    L1 = _local_layer(x, _mp(x, pos_edge_index), _mp(x, neg_edge_index),
                      l0_w, l0_b, activate=True)
    L2 = _local_layer(L1, _mp(L1, pos_edge_index), _mp(L1, neg_edge_index),
                      l1_w, l1_b, activate=False)
    return jnp.concatenate((G, L2), axis=1)


# speedup vs baseline: 1.1480x; 1.1411x over previous
import functools

import jax
import jax.numpy as jnp
from jax import lax
from jax.experimental import pallas as pl
from jax.experimental.pallas import tpu as pltpu
from jax.experimental.pallas import tpu_sc as plsc

N = 10000
E = 320000
ROW_BLK = 1000

_TILES = 16
_W = 80


def _mp(x, ei):
    return jnp.zeros(x.shape, x.dtype).at[ei[1]].add(x[ei[0]])


def _sc_mp_pair(table, src_pos, dst_pos, src_neg, dst_neg):
    n, d = table.shape
    npad = 10240
    rpt = npad // _TILES
    zr = 128
    ept = E // _TILES
    nwin = ept // _W

    mesh = plsc.VectorSubcoreMesh(core_axis_name="c", subcore_axis_name="s")

    @functools.partial(
        pl.kernel, mesh=mesh,
        out_type=jax.ShapeDtypeStruct((2, npad, d), jnp.float32),
        scratch_types=[
            pltpu.VMEM_SHARED((npad, d), jnp.float32),
            pltpu.VMEM((zr, d), jnp.float32),
            pltpu.VMEM((_W,), jnp.int32),
            pltpu.VMEM((_W,), jnp.int32),
            pltpu.VMEM((_W, d), jnp.float32),
            pltpu.SemaphoreType.DMA,
        ],
    )
    def k(tab, sp, dp, sn, dn, out, acc, zbuf, sidx, didx, rows, sem):
        c = lax.axis_index("c")
        s = lax.axis_index("s")

        zero16 = jnp.zeros((16,), jnp.float32)
        lanes_per_row = d // 16

        def zstore(i, carry):
            r = i // lanes_per_row
            j = i % lanes_per_row
            zbuf[r, pl.ds(j * 16, 16)] = zero16
            return carry

        lax.fori_loop(0, zr * lanes_per_row, zstore, 0)
        for kk in range(rpt // zr):
            pltpu.sync_copy(zbuf, acc.at[pl.ds(s * rpt + kk * zr, zr)])
        plsc.subcore_barrier()

        def run_edges(src_ref, dst_ref):
            def body(w, carry):
                base = s * ept + w * _W
                pltpu.sync_copy(src_ref.at[pl.ds(base, _W)], sidx)
                pltpu.sync_copy(dst_ref.at[pl.ds(base, _W)], didx)
                pltpu.async_copy(tab.at[sidx], rows, sem).wait()
                pltpu.sync_copy(rows, acc.at[didx], add=True)
                return carry
            lax.fori_loop(0, nwin, body, 0)

        @pl.when(c == 0)
        def _():
            run_edges(sp, dp)

        @pl.when(c == 1)
        def _():
            run_edges(sn, dn)

        plsc.subcore_barrier()
        for kk in range(rpt // zr):
            r0 = s * rpt + kk * zr
            pltpu.sync_copy(acc.at[pl.ds(r0, zr)], out.at[c, pl.ds(r0, zr)])

    return k(table, src_pos, dst_pos, src_neg, dst_neg)[:, :n, :]


def _local_body(a_ref, b_ref, c_ref, wa_ref, wb_ref, wc_ref, bias_ref, o_ref,
                *, activate):
    acc = jnp.dot(a_ref[...], wa_ref[...].T, preferred_element_type=jnp.float32)
    acc += jnp.dot(b_ref[...], wb_ref[...].T, preferred_element_type=jnp.float32)
    acc += jnp.dot(c_ref[...], wc_ref[...].T, preferred_element_type=jnp.float32)
    acc += bias_ref[...][None, :]
    if activate:
        acc = jnp.tanh(acc)
    o_ref[...] = acc


def _local_layer(a, b, c, w, bias, activate):
    d = a.shape[1]
    h = w.shape[0]
    wa, wb, wc = w[:, :d], w[:, d:2 * d], w[:, 2 * d:]
    grid = (N // ROW_BLK,)
    in_spec = pl.BlockSpec((ROW_BLK, d), lambda i: (i, 0))
    w_spec = pl.BlockSpec((h, d), lambda i: (0, 0))
    return pl.pallas_call(
        functools.partial(_local_body, activate=activate),
        grid=grid,
        in_specs=[in_spec, in_spec, in_spec, w_spec, w_spec, w_spec,
                  pl.BlockSpec((h,), lambda i: (0,))],
        out_specs=pl.BlockSpec((ROW_BLK, h), lambda i: (i, 0)),
        out_shape=jax.ShapeDtypeStruct((N, h), jnp.float32),
    )(a, b, c, wa, wb, wc, bias)


def _color_body(x_ref, w0_ref, b0_ref, w1_ref, b1_ref, cx_ref, o_ref):
    h = jnp.maximum(
        jnp.dot(x_ref[...], w0_ref[...].T, preferred_element_type=jnp.float32)
        + b0_ref[...][None, :], 0.0)
    cc = jnp.dot(h, w1_ref[...].T, preferred_element_type=jnp.float32) \
        + b1_ref[...][None, :]
    cx = jnp.dot(cc, cx_ref[...].T, preferred_element_type=jnp.float32)
    o_ref[...] = jax.nn.softmax(cx, axis=1)


def _color(x, w0, b0, w1, b1, cx):
    grid = (N // ROW_BLK,)
    return pl.pallas_call(
        _color_body,
        grid=grid,
        in_specs=[
            pl.BlockSpec((ROW_BLK, 128), lambda i: (i, 0)),
            pl.BlockSpec((64, 128), lambda i: (0, 0)),
            pl.BlockSpec((64,), lambda i: (0,)),
            pl.BlockSpec((16, 64), lambda i: (0, 0)),
            pl.BlockSpec((16,), lambda i: (0,)),
            pl.BlockSpec((3, 16), lambda i: (0, 0)),
        ],
        out_specs=pl.BlockSpec((ROW_BLK, 3), lambda i: (i, 0)),
        out_shape=jax.ShapeDtypeStruct((N, 3), jnp.float32),
    )(x, w0, b0, w1, b1, cx)


def kernel(x, pos_edge_index, neg_edge_index, color_w0, color_b0, color_w1,
           color_b1, Cx, g_lin1_w, g_lin1_b, g_lin2_w, g_lin2_b, l0_w, l0_b,
           l1_w, l1_b):
    G = _color(x, color_w0, color_b0, color_w1, color_b1, Cx)
    for i in range(5):
        g_pos = _mp(G, pos_edge_index)
        g_neg = _mp(G, neg_edge_index)
        cat = jnp.concatenate((G, g_pos, g_neg), axis=1)
        h = jnp.tanh(cat @ g_lin1_w[i].T + g_lin1_b[i])
        C = h @ g_lin2_w[i].T + g_lin2_b[i]
        G = jax.nn.softmax(C, axis=1)
    G = G @ Cx

    sp, dp = pos_edge_index[0], pos_edge_index[1]
    sn, dn = neg_edge_index[0], neg_edge_index[1]
    mm = _sc_mp_pair(x, sp, dp, sn, dn)
    L1 = _local_layer(x, mm[0], mm[1], l0_w, l0_b, activate=True)
    L1p = jnp.pad(L1, ((0, 0), (0, 16)))
    mm2 = _sc_mp_pair(L1p, sp, dp, sn, dn)
    L2 = _local_layer(L1, mm2[0, :, :112], mm2[1, :, :112],
                      l1_w, l1_b, activate=False)
    return jnp.concatenate((G, L2), axis=1)


# speedup vs baseline: 1.1713x; 1.0203x over previous
import functools

import jax
import jax.numpy as jnp
from jax import lax
from jax.experimental import pallas as pl
from jax.experimental.pallas import tpu as pltpu
from jax.experimental.pallas import tpu_sc as plsc

N = 10000
E = 320000
ROW_BLK = 1000

_TILES = 16
_W = 200


def _mp(x, ei):
    return jnp.zeros(x.shape, x.dtype).at[ei[1]].add(x[ei[0]])


def _sc_mp_pair(table, src_pos, dst_pos, src_neg, dst_neg):
    n, d = table.shape
    npad = 10240
    rpt = npad // _TILES
    zr = 64
    ept = E // _TILES
    nwin = ept // _W

    mesh = plsc.VectorSubcoreMesh(core_axis_name="c", subcore_axis_name="s")

    @functools.partial(
        pl.kernel, mesh=mesh,
        out_type=jax.ShapeDtypeStruct((2, npad, d), jnp.float32),
        scratch_types=[
            pltpu.VMEM_SHARED((npad, d), jnp.float32),
            pltpu.VMEM((zr, d), jnp.float32),
            pltpu.VMEM((_W,), jnp.int32),
            pltpu.VMEM((_W,), jnp.int32),
            pltpu.VMEM((_W, d), jnp.float32),
            pltpu.SemaphoreType.DMA,
        ],
    )
    def k(tab, sp, dp, sn, dn, out, acc, zbuf, sidx, didx, rows, sem):
        c = lax.axis_index("c")
        s = lax.axis_index("s")

        zero16 = jnp.zeros((16,), jnp.float32)
        lanes_per_row = d // 16

        def zstore(i, carry):
            r = i // lanes_per_row
            j = i % lanes_per_row
            zbuf[r, pl.ds(j * 16, 16)] = zero16
            return carry

        lax.fori_loop(0, zr * lanes_per_row, zstore, 0)
        for kk in range(rpt // zr):
            pltpu.sync_copy(zbuf, acc.at[pl.ds(s * rpt + kk * zr, zr)])
        plsc.subcore_barrier()

        def run_edges(src_ref, dst_ref):
            def body(w, carry):
                base = s * ept + w * _W
                pltpu.sync_copy(src_ref.at[pl.ds(base, _W)], sidx)
                pltpu.sync_copy(dst_ref.at[pl.ds(base, _W)], didx)
                pltpu.async_copy(tab.at[sidx], rows, sem).wait()
                pltpu.sync_copy(rows, acc.at[didx], add=True)
                return carry
            lax.fori_loop(0, nwin, body, 0)

        @pl.when(c == 0)
        def _():
            run_edges(sp, dp)

        @pl.when(c == 1)
        def _():
            run_edges(sn, dn)

        plsc.subcore_barrier()
        for kk in range(rpt // zr):
            r0 = s * rpt + kk * zr
            pltpu.sync_copy(acc.at[pl.ds(r0, zr)], out.at[c, pl.ds(r0, zr)])

    return k(table, src_pos, dst_pos, src_neg, dst_neg)[:, :n, :]


def _local_body(a_ref, b_ref, c_ref, wa_ref, wb_ref, wc_ref, bias_ref, o_ref,
                *, activate):
    acc = jnp.dot(a_ref[...], wa_ref[...].T, preferred_element_type=jnp.float32)
    acc += jnp.dot(b_ref[...], wb_ref[...].T, preferred_element_type=jnp.float32)
    acc += jnp.dot(c_ref[...], wc_ref[...].T, preferred_element_type=jnp.float32)
    acc += bias_ref[...][None, :]
    if activate:
        acc = jnp.tanh(acc)
    o_ref[...] = acc


def _local_layer(a, b, c, w, bias, activate):
    d = a.shape[1]
    h = w.shape[0]
    wa, wb, wc = w[:, :d], w[:, d:2 * d], w[:, 2 * d:]
    grid = (N // ROW_BLK,)
    in_spec = pl.BlockSpec((ROW_BLK, d), lambda i: (i, 0))
    w_spec = pl.BlockSpec((h, d), lambda i: (0, 0))
    return pl.pallas_call(
        functools.partial(_local_body, activate=activate),
        grid=grid,
        in_specs=[in_spec, in_spec, in_spec, w_spec, w_spec, w_spec,
                  pl.BlockSpec((h,), lambda i: (0,))],
        out_specs=pl.BlockSpec((ROW_BLK, h), lambda i: (i, 0)),
        out_shape=jax.ShapeDtypeStruct((N, h), jnp.float32),
    )(a, b, c, wa, wb, wc, bias)


def _color_body(x_ref, w0_ref, b0_ref, w1_ref, b1_ref, cx_ref, o_ref):
    h = jnp.maximum(
        jnp.dot(x_ref[...], w0_ref[...].T, preferred_element_type=jnp.float32)
        + b0_ref[...][None, :], 0.0)
    cc = jnp.dot(h, w1_ref[...].T, preferred_element_type=jnp.float32) \
        + b1_ref[...][None, :]
    cx = jnp.dot(cc, cx_ref[...].T, preferred_element_type=jnp.float32)
    o_ref[...] = jax.nn.softmax(cx, axis=1)


def _color(x, w0, b0, w1, b1, cx):
    grid = (N // ROW_BLK,)
    return pl.pallas_call(
        _color_body,
        grid=grid,
        in_specs=[
            pl.BlockSpec((ROW_BLK, 128), lambda i: (i, 0)),
            pl.BlockSpec((64, 128), lambda i: (0, 0)),
            pl.BlockSpec((64,), lambda i: (0,)),
            pl.BlockSpec((16, 64), lambda i: (0, 0)),
            pl.BlockSpec((16,), lambda i: (0,)),
            pl.BlockSpec((3, 16), lambda i: (0, 0)),
        ],
        out_specs=pl.BlockSpec((ROW_BLK, 3), lambda i: (i, 0)),
        out_shape=jax.ShapeDtypeStruct((N, 3), jnp.float32),
    )(x, w0, b0, w1, b1, cx)


def kernel(x, pos_edge_index, neg_edge_index, color_w0, color_b0, color_w1,
           color_b1, Cx, g_lin1_w, g_lin1_b, g_lin2_w, g_lin2_b, l0_w, l0_b,
           l1_w, l1_b):
    G = _color(x, color_w0, color_b0, color_w1, color_b1, Cx)
    for i in range(5):
        g_pos = _mp(G, pos_edge_index)
        g_neg = _mp(G, neg_edge_index)
        cat = jnp.concatenate((G, g_pos, g_neg), axis=1)
        h = jnp.tanh(cat @ g_lin1_w[i].T + g_lin1_b[i])
        C = h @ g_lin2_w[i].T + g_lin2_b[i]
        G = jax.nn.softmax(C, axis=1)
    G = G @ Cx

    sp, dp = pos_edge_index[0], pos_edge_index[1]
    sn, dn = neg_edge_index[0], neg_edge_index[1]
    mm = _sc_mp_pair(x, sp, dp, sn, dn)
    L1 = _local_layer(x, mm[0], mm[1], l0_w, l0_b, activate=True)
    L1p = jnp.pad(L1, ((0, 0), (0, 16)))
    mm2 = _sc_mp_pair(L1p, sp, dp, sn, dn)
    L2 = _local_layer(L1, mm2[0, :, :112], mm2[1, :, :112],
                      l1_w, l1_b, activate=False)
    return jnp.concatenate((G, L2), axis=1)


# speedup vs baseline: 7.7050x; 6.5781x over previous
import functools

import jax
import jax.numpy as jnp
from jax import lax
from jax.experimental import pallas as pl
from jax.experimental.pallas import tpu as pltpu
from jax.experimental.pallas import tpu_sc as plsc

N = 10000
E = 320000
ROW_BLK = 1000

_TILES = 16
_W = 200


def _mp(x, ei):
    return jnp.zeros(x.shape, x.dtype).at[ei[1]].add(x[ei[0]])


def _sc_mp_pair(table, src_pos, dst_pos, src_neg, dst_neg, w=200):
    n, d = table.shape
    npad = 10240
    rpt = npad // _TILES
    ept = E // _TILES
    nwin = ept // w
    stage = d < 128

    mesh = plsc.VectorSubcoreMesh(core_axis_name="c", subcore_axis_name="s")

    scratch = [
        pltpu.VMEM_SHARED((npad, d), jnp.float32),
        pltpu.VMEM((w,), jnp.int32),
        pltpu.VMEM((w,), jnp.int32),
        pltpu.VMEM((w, d), jnp.float32),
        pltpu.SemaphoreType.DMA,
    ]
    if stage:
        scratch.append(pltpu.VMEM_SHARED((npad, d), jnp.float32))

    @functools.partial(
        pl.kernel, mesh=mesh,
        out_type=jax.ShapeDtypeStruct((2, npad, d), jnp.float32),
        scratch_types=scratch,
    )
    def k(tab, sp, dp, sn, dn, zh, out, acc, sidx, didx, rows, sem,
          *maybe_tspm):
        c = lax.axis_index("c")
        s = lax.axis_index("s")
        r0 = s * rpt

        pltpu.sync_copy(zh, acc.at[pl.ds(r0, rpt)])
        if stage:
            tspm = maybe_tspm[0]
            pltpu.sync_copy(tab.at[pl.ds(r0, rpt)], tspm.at[pl.ds(r0, rpt)])
            gsrc = tspm
        else:
            gsrc = tab
        plsc.subcore_barrier()

        def run_edges(src_ref, dst_ref):
            def body(wi, carry):
                base = s * ept + wi * w
                pltpu.sync_copy(src_ref.at[pl.ds(base, w)], sidx)
                pltpu.sync_copy(dst_ref.at[pl.ds(base, w)], didx)
                pltpu.async_copy(gsrc.at[sidx], rows, sem).wait()
                pltpu.sync_copy(rows, acc.at[didx], add=True)
                return carry
            lax.fori_loop(0, nwin, body, 0)

        @pl.when(c == 0)
        def _():
            run_edges(sp, dp)

        @pl.when(c == 1)
        def _():
            run_edges(sn, dn)

        plsc.subcore_barrier()
        pltpu.sync_copy(acc.at[pl.ds(r0, rpt)], out.at[c, pl.ds(r0, rpt)])

    zrows = jnp.zeros((rpt, d), jnp.float32)
    return k(table, src_pos, dst_pos, src_neg, dst_neg, zrows)[:, :n, :]


def _local_body(a_ref, b_ref, c_ref, wa_ref, wb_ref, wc_ref, bias_ref, o_ref,
                *, activate):
    acc = jnp.dot(a_ref[...], wa_ref[...].T, preferred_element_type=jnp.float32)
    acc += jnp.dot(b_ref[...], wb_ref[...].T, preferred_element_type=jnp.float32)
    acc += jnp.dot(c_ref[...], wc_ref[...].T, preferred_element_type=jnp.float32)
    acc += bias_ref[...][None, :]
    if activate:
        acc = jnp.tanh(acc)
    o_ref[...] = acc


def _local_layer(a, b, c, w, bias, activate):
    d = a.shape[1]
    h = w.shape[0]
    wa, wb, wc = w[:, :d], w[:, d:2 * d], w[:, 2 * d:]
    grid = (N // ROW_BLK,)
    in_spec = pl.BlockSpec((ROW_BLK, d), lambda i: (i, 0))
    w_spec = pl.BlockSpec((h, d), lambda i: (0, 0))
    return pl.pallas_call(
        functools.partial(_local_body, activate=activate),
        grid=grid,
        in_specs=[in_spec, in_spec, in_spec, w_spec, w_spec, w_spec,
                  pl.BlockSpec((h,), lambda i: (0,))],
        out_specs=pl.BlockSpec((ROW_BLK, h), lambda i: (i, 0)),
        out_shape=jax.ShapeDtypeStruct((N, h), jnp.float32),
    )(a, b, c, wa, wb, wc, bias)


def _gmlp_body(g_ref, p_ref, n_ref, w1_ref, b1_ref, w2_ref, b2_ref, o_ref):
    w1 = w1_ref[...]
    h = jnp.dot(g_ref[...], w1[:, :3].T, preferred_element_type=jnp.float32)
    h += jnp.dot(p_ref[...], w1[:, 3:6].T, preferred_element_type=jnp.float32)
    h += jnp.dot(n_ref[...], w1[:, 6:9].T, preferred_element_type=jnp.float32)
    h = jnp.tanh(h + b1_ref[...][None, :])
    c = jnp.dot(h, w2_ref[...].T, preferred_element_type=jnp.float32) \
        + b2_ref[...][None, :]
    o_ref[...] = jax.nn.softmax(c, axis=1)


def _gmlp(g, p, n, w1, b1, w2, b2):
    grid = (N // ROW_BLK,)
    in_spec = pl.BlockSpec((ROW_BLK, 3), lambda i: (i, 0))
    return pl.pallas_call(
        _gmlp_body,
        grid=grid,
        in_specs=[in_spec, in_spec, in_spec,
                  pl.BlockSpec((16, 9), lambda i: (0, 0)),
                  pl.BlockSpec((16,), lambda i: (0,)),
                  pl.BlockSpec((3, 16), lambda i: (0, 0)),
                  pl.BlockSpec((3,), lambda i: (0,))],
        out_specs=pl.BlockSpec((ROW_BLK, 3), lambda i: (i, 0)),
        out_shape=jax.ShapeDtypeStruct((N, 3), jnp.float32),
    )(g, p, n, w1, b1, w2, b2)


def _color_body(x_ref, w0_ref, b0_ref, w1_ref, b1_ref, cx_ref, o_ref):
    h = jnp.maximum(
        jnp.dot(x_ref[...], w0_ref[...].T, preferred_element_type=jnp.float32)
        + b0_ref[...][None, :], 0.0)
    cc = jnp.dot(h, w1_ref[...].T, preferred_element_type=jnp.float32) \
        + b1_ref[...][None, :]
    cx = jnp.dot(cc, cx_ref[...].T, preferred_element_type=jnp.float32)
    o_ref[...] = jax.nn.softmax(cx, axis=1)


def _color(x, w0, b0, w1, b1, cx):
    grid = (N // ROW_BLK,)
    return pl.pallas_call(
        _color_body,
        grid=grid,
        in_specs=[
            pl.BlockSpec((ROW_BLK, 128), lambda i: (i, 0)),
            pl.BlockSpec((64, 128), lambda i: (0, 0)),
            pl.BlockSpec((64,), lambda i: (0,)),
            pl.BlockSpec((16, 64), lambda i: (0, 0)),
            pl.BlockSpec((16,), lambda i: (0,)),
            pl.BlockSpec((3, 16), lambda i: (0, 0)),
        ],
        out_specs=pl.BlockSpec((ROW_BLK, 3), lambda i: (i, 0)),
        out_shape=jax.ShapeDtypeStruct((N, 3), jnp.float32),
    )(x, w0, b0, w1, b1, cx)


def kernel(x, pos_edge_index, neg_edge_index, color_w0, color_b0, color_w1,
           color_b1, Cx, g_lin1_w, g_lin1_b, g_lin2_w, g_lin2_b, l0_w, l0_b,
           l1_w, l1_b):
    sp, dp = pos_edge_index[0], pos_edge_index[1]
    sn, dn = neg_edge_index[0], neg_edge_index[1]

    G = _color(x, color_w0, color_b0, color_w1, color_b1, Cx)
    for i in range(5):
        gpad = jnp.pad(G, ((0, 240), (0, 5)))
        mg = _sc_mp_pair(gpad, sp, dp, sn, dn, w=200)
        G = _gmlp(G, mg[0, :N, :3], mg[1, :N, :3],
                  g_lin1_w[i], g_lin1_b[i], g_lin2_w[i], g_lin2_b[i])
    G = G @ Cx
    mm = _sc_mp_pair(x, sp, dp, sn, dn)
    L1 = _local_layer(x, mm[0], mm[1], l0_w, l0_b, activate=True)
    L1p = jnp.pad(L1, ((0, 0), (0, 16)))
    mm2 = _sc_mp_pair(L1p, sp, dp, sn, dn)
    L2 = _local_layer(L1, mm2[0, :, :112], mm2[1, :, :112],
                      l1_w, l1_b, activate=False)
    return jnp.concatenate((G, L2), axis=1)


# speedup vs baseline: 10.6788x; 1.3860x over previous
import functools

import jax
import jax.numpy as jnp
from jax import lax
from jax.experimental import pallas as pl
from jax.experimental.pallas import tpu as pltpu
from jax.experimental.pallas import tpu_sc as plsc

N = 10000
E = 320000
ROW_BLK = 1000

_TILES = 16
_W = 200


def _mp(x, ei):
    return jnp.zeros(x.shape, x.dtype).at[ei[1]].add(x[ei[0]])


def _sc_mp_pair(table, src_pos, dst_pos, src_neg, dst_neg, w=200, ck=5):
    n, d = table.shape
    npad = 10240
    rpt = npad // _TILES
    ept = E // _TILES
    nwin = ept // w
    nchunk = nwin // ck
    assert nchunk * ck * w == ept
    stage = d < 128

    mesh = plsc.VectorSubcoreMesh(core_axis_name="c", subcore_axis_name="s")

    scratch = [
        pltpu.VMEM_SHARED((npad, d), jnp.float32),
        pltpu.VMEM((ck, 1, w), jnp.int32),
        pltpu.VMEM((ck, 1, w), jnp.int32),
        pltpu.VMEM((w, d), jnp.float32),
        pltpu.SemaphoreType.DMA,
    ]
    if stage:
        scratch.append(pltpu.VMEM_SHARED((npad, d), jnp.float32))

    @functools.partial(
        pl.kernel, mesh=mesh,
        out_type=jax.ShapeDtypeStruct((2, npad, d), jnp.float32),
        scratch_types=scratch,
    )
    def k(tab, sp, dp, sn, dn, zh, out, acc, sidx, didx, rows, gsem,
          *maybe_tspm):
        c = lax.axis_index("c")
        s = lax.axis_index("s")
        r0 = s * rpt

        pltpu.sync_copy(zh, acc.at[pl.ds(r0, rpt)])
        if stage:
            tspm = maybe_tspm[0]
            pltpu.sync_copy(tab.at[pl.ds(r0, rpt)], tspm.at[pl.ds(r0, rpt)])
            gsrc = tspm
        else:
            gsrc = tab
        plsc.subcore_barrier()

        def run_edges(src_ref, dst_ref):
            def chunk(ci, carry):
                pltpu.sync_copy(src_ref.at[s, ci], sidx)
                pltpu.sync_copy(dst_ref.at[s, ci], didx)
                for kk in range(ck):
                    pltpu.async_copy(gsrc.at[sidx.at[kk, 0]], rows, gsem).wait()
                    pltpu.sync_copy(rows, acc.at[didx.at[kk, 0]], add=True)
                return carry
            lax.fori_loop(0, nchunk, chunk, 0)

        @pl.when(c == 0)
        def _():
            run_edges(sp, dp)

        @pl.when(c == 1)
        def _():
            run_edges(sn, dn)

        plsc.subcore_barrier()
        pltpu.sync_copy(acc.at[pl.ds(r0, rpt)], out.at[c, pl.ds(r0, rpt)])

    zrows = jnp.zeros((rpt, d), jnp.float32)
    shp = (_TILES, nchunk, ck, 1, w)
    return k(table, src_pos.reshape(shp), dst_pos.reshape(shp),
             src_neg.reshape(shp), dst_neg.reshape(shp),
             zrows)[:, :n, :]


def _local_body(a_ref, b_ref, c_ref, wa_ref, wb_ref, wc_ref, bias_ref, o_ref,
                *, activate):
    acc = jnp.dot(a_ref[...], wa_ref[...].T, preferred_element_type=jnp.float32)
    acc += jnp.dot(b_ref[...], wb_ref[...].T, preferred_element_type=jnp.float32)
    acc += jnp.dot(c_ref[...], wc_ref[...].T, preferred_element_type=jnp.float32)
    acc += bias_ref[...][None, :]
    if activate:
        acc = jnp.tanh(acc)
    o_ref[...] = acc


def _local_layer(a, b, c, w, bias, activate):
    d = a.shape[1]
    h = w.shape[0]
    wa, wb, wc = w[:, :d], w[:, d:2 * d], w[:, 2 * d:]
    grid = (N // ROW_BLK,)
    in_spec = pl.BlockSpec((ROW_BLK, d), lambda i: (i, 0))
    w_spec = pl.BlockSpec((h, d), lambda i: (0, 0))
    return pl.pallas_call(
        functools.partial(_local_body, activate=activate),
        grid=grid,
        in_specs=[in_spec, in_spec, in_spec, w_spec, w_spec, w_spec,
                  pl.BlockSpec((h,), lambda i: (0,))],
        out_specs=pl.BlockSpec((ROW_BLK, h), lambda i: (i, 0)),
        out_shape=jax.ShapeDtypeStruct((N, h), jnp.float32),
    )(a, b, c, wa, wb, wc, bias)


def _gmlp_body(g_ref, p_ref, n_ref, w1_ref, b1_ref, w2_ref, b2_ref, o_ref):
    w1 = w1_ref[...]
    h = jnp.dot(g_ref[...], w1[:, :3].T, preferred_element_type=jnp.float32)
    h += jnp.dot(p_ref[...], w1[:, 3:6].T, preferred_element_type=jnp.float32)
    h += jnp.dot(n_ref[...], w1[:, 6:9].T, preferred_element_type=jnp.float32)
    h = jnp.tanh(h + b1_ref[...][None, :])
    c = jnp.dot(h, w2_ref[...].T, preferred_element_type=jnp.float32) \
        + b2_ref[...][None, :]
    o_ref[...] = jax.nn.softmax(c, axis=1)


def _gmlp(g, p, n, w1, b1, w2, b2):
    grid = (N // ROW_BLK,)
    in_spec = pl.BlockSpec((ROW_BLK, 3), lambda i: (i, 0))
    return pl.pallas_call(
        _gmlp_body,
        grid=grid,
        in_specs=[in_spec, in_spec, in_spec,
                  pl.BlockSpec((16, 9), lambda i: (0, 0)),
                  pl.BlockSpec((16,), lambda i: (0,)),
                  pl.BlockSpec((3, 16), lambda i: (0, 0)),
                  pl.BlockSpec((3,), lambda i: (0,))],
        out_specs=pl.BlockSpec((ROW_BLK, 3), lambda i: (i, 0)),
        out_shape=jax.ShapeDtypeStruct((N, 3), jnp.float32),
    )(g, p, n, w1, b1, w2, b2)


def _color_body(x_ref, w0_ref, b0_ref, w1_ref, b1_ref, cx_ref, o_ref):
    h = jnp.maximum(
        jnp.dot(x_ref[...], w0_ref[...].T, preferred_element_type=jnp.float32)
        + b0_ref[...][None, :], 0.0)
    cc = jnp.dot(h, w1_ref[...].T, preferred_element_type=jnp.float32) \
        + b1_ref[...][None, :]
    cx = jnp.dot(cc, cx_ref[...].T, preferred_element_type=jnp.float32)
    o_ref[...] = jax.nn.softmax(cx, axis=1)


def _color(x, w0, b0, w1, b1, cx):
    grid = (N // ROW_BLK,)
    return pl.pallas_call(
        _color_body,
        grid=grid,
        in_specs=[
            pl.BlockSpec((ROW_BLK, 128), lambda i: (i, 0)),
            pl.BlockSpec((64, 128), lambda i: (0, 0)),
            pl.BlockSpec((64,), lambda i: (0,)),
            pl.BlockSpec((16, 64), lambda i: (0, 0)),
            pl.BlockSpec((16,), lambda i: (0,)),
            pl.BlockSpec((3, 16), lambda i: (0, 0)),
        ],
        out_specs=pl.BlockSpec((ROW_BLK, 3), lambda i: (i, 0)),
        out_shape=jax.ShapeDtypeStruct((N, 3), jnp.float32),
    )(x, w0, b0, w1, b1, cx)


def kernel(x, pos_edge_index, neg_edge_index, color_w0, color_b0, color_w1,
           color_b1, Cx, g_lin1_w, g_lin1_b, g_lin2_w, g_lin2_b, l0_w, l0_b,
           l1_w, l1_b):
    sp, dp = pos_edge_index[0], pos_edge_index[1]
    sn, dn = neg_edge_index[0], neg_edge_index[1]

    G = _color(x, color_w0, color_b0, color_w1, color_b1, Cx)
    for i in range(5):
        gpad = jnp.pad(G, ((0, 240), (0, 5)))
        mg = _sc_mp_pair(gpad, sp, dp, sn, dn, w=200, ck=10)
        G = _gmlp(G, mg[0, :N, :3], mg[1, :N, :3],
                  g_lin1_w[i], g_lin1_b[i], g_lin2_w[i], g_lin2_b[i])
    G = G @ Cx
    mm = _sc_mp_pair(x, sp, dp, sn, dn, w=160, ck=5)
    L1 = _local_layer(x, mm[0], mm[1], l0_w, l0_b, activate=True)
    L1p = jnp.pad(L1, ((0, 0), (0, 16)))
    mm2 = _sc_mp_pair(L1p, sp, dp, sn, dn, w=160, ck=5)
    L2 = _local_layer(L1, mm2[0, :, :112], mm2[1, :, :112],
                      l1_w, l1_b, activate=False)
    return jnp.concatenate((G, L2), axis=1)


# speedup vs baseline: 11.2855x; 1.0568x over previous
import functools

import jax
import jax.numpy as jnp
from jax import lax
from jax.experimental import pallas as pl
from jax.experimental.pallas import tpu as pltpu
from jax.experimental.pallas import tpu_sc as plsc

N = 10000
E = 320000
ROW_BLK = 1000

_TILES = 16
_W = 200


def _mp(x, ei):
    return jnp.zeros(x.shape, x.dtype).at[ei[1]].add(x[ei[0]])


def _sc_mp_pair(table, src_pos, dst_pos, src_neg, dst_neg, w=200, ck=5):
    n, d = table.shape
    npad = 10240
    rpt = npad // _TILES
    ept = E // _TILES
    nwin = ept // w
    nchunk = nwin // ck
    assert nchunk * ck * w == ept
    stage = d < 128

    mesh = plsc.VectorSubcoreMesh(core_axis_name="c", subcore_axis_name="s")

    scratch = [
        pltpu.VMEM_SHARED((npad, d), jnp.float32),
        pltpu.VMEM((ck, 1, w), jnp.int32),
        pltpu.VMEM((ck, 1, w), jnp.int32),
        pltpu.VMEM((w, d), jnp.float32),
        pltpu.SemaphoreType.DMA,
    ]
    if stage:
        scratch.append(pltpu.VMEM_SHARED((npad, d), jnp.float32))

    @functools.partial(
        pl.kernel, mesh=mesh,
        out_type=jax.ShapeDtypeStruct((2, npad, d), jnp.float32),
        scratch_types=scratch,
    )
    def k(tab, sp, dp, sn, dn, zh, out, acc, sidx, didx, rows, gsem,
          *maybe_tspm):
        c = lax.axis_index("c")
        s = lax.axis_index("s")
        r0 = s * rpt

        pltpu.sync_copy(zh, acc.at[pl.ds(r0, rpt)])
        if stage:
            tspm = maybe_tspm[0]
            pltpu.sync_copy(tab.at[pl.ds(r0, rpt)], tspm.at[pl.ds(r0, rpt)])
            gsrc = tspm
        else:
            gsrc = tab
        plsc.subcore_barrier()

        def run_edges(src_ref, dst_ref):
            def chunk(ci, carry):
                pltpu.sync_copy(src_ref.at[s, ci], sidx)
                pltpu.sync_copy(dst_ref.at[s, ci], didx)
                for kk in range(ck):
                    pltpu.async_copy(gsrc.at[sidx.at[kk, 0]], rows, gsem).wait()
                    pltpu.sync_copy(rows, acc.at[didx.at[kk, 0]], add=True)
                return carry
            lax.fori_loop(0, nchunk, chunk, 0)

        @pl.when(c == 0)
        def _():
            run_edges(sp, dp)

        @pl.when(c == 1)
        def _():
            run_edges(sn, dn)

        plsc.subcore_barrier()
        pltpu.sync_copy(acc.at[pl.ds(r0, rpt)], out.at[c, pl.ds(r0, rpt)])

    zrows = jnp.zeros((rpt, d), jnp.float32)
    shp = (_TILES, nchunk, ck, 1, w)
    return k(table, src_pos.reshape(shp), dst_pos.reshape(shp),
             src_neg.reshape(shp), dst_neg.reshape(shp),
             zrows)[:, :n, :]


def _local_body(a_ref, b_ref, c_ref, wa_ref, wb_ref, wc_ref, bias_ref, o_ref,
                *, activate):
    acc = jnp.dot(a_ref[...], wa_ref[...].T, preferred_element_type=jnp.float32)
    acc += jnp.dot(b_ref[...], wb_ref[...].T, preferred_element_type=jnp.float32)
    acc += jnp.dot(c_ref[...], wc_ref[...].T, preferred_element_type=jnp.float32)
    acc += bias_ref[...][None, :]
    if activate:
        acc = jnp.tanh(acc)
    o_ref[...] = acc


def _local_layer(a, b, c, w, bias, activate):
    d = a.shape[1]
    h = w.shape[0]
    wa, wb, wc = w[:, :d], w[:, d:2 * d], w[:, 2 * d:]
    grid = (N // ROW_BLK,)
    in_spec = pl.BlockSpec((ROW_BLK, d), lambda i: (i, 0))
    w_spec = pl.BlockSpec((h, d), lambda i: (0, 0))
    return pl.pallas_call(
        functools.partial(_local_body, activate=activate),
        grid=grid,
        in_specs=[in_spec, in_spec, in_spec, w_spec, w_spec, w_spec,
                  pl.BlockSpec((h,), lambda i: (0,))],
        out_specs=pl.BlockSpec((ROW_BLK, h), lambda i: (i, 0)),
        out_shape=jax.ShapeDtypeStruct((N, h), jnp.float32),
    )(a, b, c, wa, wb, wc, bias)


def _gmlp_body(g_ref, p_ref, n_ref, w1_ref, b1_ref, w2_ref, b2_ref, o_ref):
    w1 = w1_ref[...]
    g3, p3, n3 = g_ref[...][:, :3], p_ref[...][:, :3], n_ref[...][:, :3]
    h = jnp.dot(g3, w1[:, :3].T, preferred_element_type=jnp.float32)
    h += jnp.dot(p3, w1[:, 3:6].T, preferred_element_type=jnp.float32)
    h += jnp.dot(n3, w1[:, 6:9].T, preferred_element_type=jnp.float32)
    h = jnp.tanh(h + b1_ref[...][None, :])
    c = jnp.dot(h, w2_ref[...].T, preferred_element_type=jnp.float32) \
        + b2_ref[...][None, :]
    sm = jax.nn.softmax(c, axis=1)
    o_ref[...] = jnp.pad(sm, ((0, 0), (0, 5)))


def _gmlp(g, p, n, w1, b1, w2, b2):
    blk = 1024
    grid = (10240 // blk,)
    in_spec = pl.BlockSpec((blk, 8), lambda i: (i, 0))
    return pl.pallas_call(
        _gmlp_body,
        grid=grid,
        in_specs=[in_spec, in_spec, in_spec,
                  pl.BlockSpec((16, 9), lambda i: (0, 0)),
                  pl.BlockSpec((16,), lambda i: (0,)),
                  pl.BlockSpec((3, 16), lambda i: (0, 0)),
                  pl.BlockSpec((3,), lambda i: (0,))],
        out_specs=pl.BlockSpec((blk, 8), lambda i: (i, 0)),
        out_shape=jax.ShapeDtypeStruct((10240, 8), jnp.float32),
    )(g, p, n, w1, b1, w2, b2)


def _color_body(x_ref, w0_ref, b0_ref, w1_ref, b1_ref, cx_ref, o_ref):
    h = jnp.maximum(
        jnp.dot(x_ref[...], w0_ref[...].T, preferred_element_type=jnp.float32)
        + b0_ref[...][None, :], 0.0)
    cc = jnp.dot(h, w1_ref[...].T, preferred_element_type=jnp.float32) \
        + b1_ref[...][None, :]
    cx = jnp.dot(cc, cx_ref[...].T, preferred_element_type=jnp.float32)
    o_ref[...] = jax.nn.softmax(cx, axis=1)


def _color(x, w0, b0, w1, b1, cx):
    grid = (N // ROW_BLK,)
    return pl.pallas_call(
        _color_body,
        grid=grid,
        in_specs=[
            pl.BlockSpec((ROW_BLK, 128), lambda i: (i, 0)),
            pl.BlockSpec((64, 128), lambda i: (0, 0)),
            pl.BlockSpec((64,), lambda i: (0,)),
            pl.BlockSpec((16, 64), lambda i: (0, 0)),
            pl.BlockSpec((16,), lambda i: (0,)),
            pl.BlockSpec((3, 16), lambda i: (0, 0)),
        ],
        out_specs=pl.BlockSpec((ROW_BLK, 3), lambda i: (i, 0)),
        out_shape=jax.ShapeDtypeStruct((N, 3), jnp.float32),
    )(x, w0, b0, w1, b1, cx)


def kernel(x, pos_edge_index, neg_edge_index, color_w0, color_b0, color_w1,
           color_b1, Cx, g_lin1_w, g_lin1_b, g_lin2_w, g_lin2_b, l0_w, l0_b,
           l1_w, l1_b):
    sp, dp = pos_edge_index[0], pos_edge_index[1]
    sn, dn = neg_edge_index[0], neg_edge_index[1]

    G = jnp.pad(_color(x, color_w0, color_b0, color_w1, color_b1, Cx),
                ((0, 240), (0, 5)))
    for i in range(5):
        mg = _sc_mp_pair(G, sp, dp, sn, dn, w=200, ck=10)
        G = _gmlp(G, mg[0], mg[1],
                  g_lin1_w[i], g_lin1_b[i], g_lin2_w[i], g_lin2_b[i])
    G = G[:N, :3] @ Cx
    mm = _sc_mp_pair(x, sp, dp, sn, dn, w=200, ck=5)
    L1 = _local_layer(x, mm[0], mm[1], l0_w, l0_b, activate=True)
    L1p = jnp.pad(L1, ((0, 0), (0, 16)))
    mm2 = _sc_mp_pair(L1p, sp, dp, sn, dn, w=200, ck=5)
    L2 = _local_layer(L1, mm2[0, :, :112], mm2[1, :, :112],
                      l1_w, l1_b, activate=False)
    return jnp.concatenate((G, L2), axis=1)


# speedup vs baseline: 11.7283x; 1.0392x over previous
import functools

import jax
import jax.numpy as jnp
from jax import lax
from jax.experimental import pallas as pl
from jax.experimental.pallas import tpu as pltpu
from jax.experimental.pallas import tpu_sc as plsc

N = 10000
E = 320000
ROW_BLK = 1000

_TILES = 16


def _sc_mp_pair(table, src_pos, dst_pos, src_neg, dst_neg, w=200, ck=5):
    n, d = table.shape
    npad = 10240
    rpt = npad // _TILES
    ept = E // _TILES
    nwin = ept // w
    nchunk = nwin // ck
    assert nchunk * ck * w == ept
    stage = d < 128

    mesh = plsc.VectorSubcoreMesh(core_axis_name="c", subcore_axis_name="s")

    scratch = [
        pltpu.VMEM_SHARED((npad, d), jnp.float32),
        pltpu.VMEM((ck, 1, w), jnp.int32),
        pltpu.VMEM((ck, 1, w), jnp.int32),
        pltpu.VMEM((w, d), jnp.float32),
        pltpu.SemaphoreType.DMA,
    ]
    if stage:
        scratch.append(pltpu.VMEM_SHARED((npad, d), jnp.float32))

    @functools.partial(
        pl.kernel, mesh=mesh,
        out_type=jax.ShapeDtypeStruct((2, npad, d), jnp.float32),
        scratch_types=scratch,
    )
    def k(tab, sp, dp, sn, dn, zh, out, acc, sidx, didx, rows, gsem,
          *maybe_tspm):
        c = lax.axis_index("c")
        s = lax.axis_index("s")
        r0 = s * rpt

        pltpu.sync_copy(zh, acc.at[pl.ds(r0, rpt)])
        if stage:
            tspm = maybe_tspm[0]
            pltpu.sync_copy(tab.at[pl.ds(r0, rpt)], tspm.at[pl.ds(r0, rpt)])
            gsrc = tspm
        else:
            gsrc = tab
        plsc.subcore_barrier()

        def run_edges(src_ref, dst_ref):
            def chunk(ci, carry):
                pltpu.sync_copy(src_ref.at[s, ci], sidx)
                pltpu.sync_copy(dst_ref.at[s, ci], didx)
                for kk in range(ck):
                    pltpu.async_copy(gsrc.at[sidx.at[kk, 0]], rows, gsem).wait()
                    pltpu.sync_copy(rows, acc.at[didx.at[kk, 0]], add=True)
                return carry
            lax.fori_loop(0, nchunk, chunk, 0)

        @pl.when(c == 0)
        def _():
            run_edges(sp, dp)

        @pl.when(c == 1)
        def _():
            run_edges(sn, dn)

        plsc.subcore_barrier()
        pltpu.sync_copy(acc.at[pl.ds(r0, rpt)], out.at[c, pl.ds(r0, rpt)])

    zrows = jnp.zeros((rpt, d), jnp.float32)
    shp = (_TILES, nchunk, ck, 1, w)
    return k(table, src_pos.reshape(shp), dst_pos.reshape(shp),
             src_neg.reshape(shp), dst_neg.reshape(shp),
             zrows)[:, :n, :]


def _local_body(a_ref, b_ref, c_ref, wa_ref, wb_ref, wc_ref, bias_ref, o_ref,
                *, activate):
    acc = jnp.dot(a_ref[...], wa_ref[...].T, preferred_element_type=jnp.float32)
    acc += jnp.dot(b_ref[...], wb_ref[...].T, preferred_element_type=jnp.float32)
    acc += jnp.dot(c_ref[...], wc_ref[...].T, preferred_element_type=jnp.float32)
    acc += bias_ref[...][None, :]
    if activate:
        acc = jnp.tanh(acc)
    o_ref[...] = acc


def _local_layer(a, b, c, w, bias, activate):
    d = a.shape[1]
    h = w.shape[0]
    wa, wb, wc = w[:, :d], w[:, d:2 * d], w[:, 2 * d:]
    grid = (N // ROW_BLK,)
    in_spec = pl.BlockSpec((ROW_BLK, d), lambda i: (i, 0))
    w_spec = pl.BlockSpec((h, d), lambda i: (0, 0))
    return pl.pallas_call(
        functools.partial(_local_body, activate=activate),
        grid=grid,
        in_specs=[in_spec, in_spec, in_spec, w_spec, w_spec, w_spec,
                  pl.BlockSpec((h,), lambda i: (0,))],
        out_specs=pl.BlockSpec((ROW_BLK, h), lambda i: (i, 0)),
        out_shape=jax.ShapeDtypeStruct((N, h), jnp.float32),
    )(a, b, c, wa, wb, wc, bias)


def _gmlp_body(g_ref, p_ref, n_ref, w1_ref, b1_ref, w2_ref, b2_ref, o_ref):
    w1 = w1_ref[...]
    g3, p3, n3 = g_ref[...][:, :3], p_ref[...][:, :3], n_ref[...][:, :3]
    h = jnp.dot(g3, w1[:, :3].T, preferred_element_type=jnp.float32)
    h += jnp.dot(p3, w1[:, 3:6].T, preferred_element_type=jnp.float32)
    h += jnp.dot(n3, w1[:, 6:9].T, preferred_element_type=jnp.float32)
    h = jnp.tanh(h + b1_ref[...][None, :])
    c = jnp.dot(h, w2_ref[...].T, preferred_element_type=jnp.float32) \
        + b2_ref[...][None, :]
    sm = jax.nn.softmax(c, axis=1)
    o_ref[...] = jnp.pad(sm, ((0, 0), (0, 5)))


def _gmlp(g, p, n, w1, b1, w2, b2):
    blk = 1024
    grid = (10240 // blk,)
    in_spec = pl.BlockSpec((blk, 8), lambda i: (i, 0))
    return pl.pallas_call(
        _gmlp_body,
        grid=grid,
        in_specs=[in_spec, in_spec, in_spec,
                  pl.BlockSpec((16, 9), lambda i: (0, 0)),
                  pl.BlockSpec((16,), lambda i: (0,)),
                  pl.BlockSpec((3, 16), lambda i: (0, 0)),
                  pl.BlockSpec((3,), lambda i: (0,))],
        out_specs=pl.BlockSpec((blk, 8), lambda i: (i, 0)),
        out_shape=jax.ShapeDtypeStruct((10240, 8), jnp.float32),
    )(g, p, n, w1, b1, w2, b2)


def _color_body(x_ref, w0_ref, b0_ref, w1_ref, b1_ref, cx_ref, o_ref):
    h = jnp.maximum(
        jnp.dot(x_ref[...], w0_ref[...].T, preferred_element_type=jnp.float32)
        + b0_ref[...][None, :], 0.0)
    cc = jnp.dot(h, w1_ref[...].T, preferred_element_type=jnp.float32) \
        + b1_ref[...][None, :]
    cx = jnp.dot(cc, cx_ref[...].T, preferred_element_type=jnp.float32)
    o_ref[...] = jax.nn.softmax(cx, axis=1)


def _color(x, w0, b0, w1, b1, cx):
    grid = (N // ROW_BLK,)
    return pl.pallas_call(
        _color_body,
        grid=grid,
        in_specs=[
            pl.BlockSpec((ROW_BLK, 128), lambda i: (i, 0)),
            pl.BlockSpec((64, 128), lambda i: (0, 0)),
            pl.BlockSpec((64,), lambda i: (0,)),
            pl.BlockSpec((16, 64), lambda i: (0, 0)),
            pl.BlockSpec((16,), lambda i: (0,)),
            pl.BlockSpec((3, 16), lambda i: (0, 0)),
        ],
        out_specs=pl.BlockSpec((ROW_BLK, 3), lambda i: (i, 0)),
        out_shape=jax.ShapeDtypeStruct((N, 3), jnp.float32),
    )(x, w0, b0, w1, b1, cx)


def kernel(x, pos_edge_index, neg_edge_index, color_w0, color_b0, color_w1,
           color_b1, Cx, g_lin1_w, g_lin1_b, g_lin2_w, g_lin2_b, l0_w, l0_b,
           l1_w, l1_b):
    sp, dp = pos_edge_index[0], pos_edge_index[1]
    sn, dn = neg_edge_index[0], neg_edge_index[1]

    G = jnp.pad(_color(x, color_w0, color_b0, color_w1, color_b1, Cx),
                ((0, 240), (0, 5)))
    for i in range(5):
        mg = _sc_mp_pair(G, sp, dp, sn, dn, w=200, ck=20)
        G = _gmlp(G, mg[0], mg[1],
                  g_lin1_w[i], g_lin1_b[i], g_lin2_w[i], g_lin2_b[i])
    G = G[:N, :3] @ Cx
    mm = _sc_mp_pair(x, sp, dp, sn, dn, w=200, ck=10)
    L1 = _local_layer(x, mm[0], mm[1], l0_w, l0_b, activate=True)
    L1p = jnp.pad(L1, ((0, 0), (0, 16)))
    mm2 = _sc_mp_pair(L1p, sp, dp, sn, dn, w=200, ck=10)
    L2 = _local_layer(L1, mm2[0, :, :112], mm2[1, :, :112],
                      l1_w, l1_b, activate=False)
    return jnp.concatenate((G, L2), axis=1)
